# baseline (speedup 1.0000x reference)
import jax
import jax.numpy as jnp
import numpy as np
from jax.experimental import pallas as pl

B = 1024
V = 64
TDIM = 128
G = 64
GV = G * V
F32 = jnp.float32
BF16 = jnp.bfloat16


def _mm(a, b, out_dtype=F32):
    return jax.lax.dot_general(a, b, (((1,), (0,)), ((), ())),
                               preferred_element_type=out_dtype)


def _mm_t(a, b, out_dtype=F32):
    return jax.lax.dot_general(a, b, (((0,), (1,)), ((), ())),
                               preferred_element_type=out_dtype)


def _mm_tl(a, b, out_dtype=F32):
    return jax.lax.dot_general(a, b, (((0,), (0,)), ((), ())),
                               preferred_element_type=out_dtype)


def _leaky(x):
    return jnp.where(x >= 0, x, 0.2 * x)


def _silu(x):
    return x * jax.lax.logistic(x)


def _elu(x):
    return jnp.where(x > 0, x, jnp.exp(jnp.minimum(x, jnp.zeros_like(x))) - 1)


def _roll_prev(x3):
    return jnp.concatenate([x3[:, V - 1:, :], x3[:, :V - 1, :]], axis=1)


def _roll_next(x3):
    return jnp.concatenate([x3[:, 1:, :], x3[:, :1, :]], axis=1)


def _lroll(x, k):
    return jnp.concatenate([x[:, k:], x[:, :k]], axis=1)


def _attn_w(ssT, stT, m0, m63):
    prv = jnp.where(m0, _lroll(ssT, V - 1), _lroll(ssT, GV - 1))
    nxt = jnp.where(m63, _lroll(ssT, GV - (V - 1)), _lroll(ssT, 1))
    e_s = _leaky(ssT + stT)
    e_p = _leaky(prv + stT)
    e_n = _leaky(nxt + stT)
    m = jnp.maximum(e_s, jnp.maximum(e_p, e_n))
    x_s = jnp.exp(e_s - m)
    x_p = jnp.exp(e_p - m)
    x_n = jnp.exp(e_n - m)
    den = x_s + x_p + x_n + 1e-16
    return x_p / den, x_n / den


def _combine(proj, skip, w_pT, w_nT, bias, E, act):
    w_p = _mm_tl(w_pT.astype(BF16), E)
    w_n = _mm_tl(w_nT.astype(BF16), E)
    p3 = proj.reshape(G, V, 256)
    p_p = _roll_prev(p3).reshape(GV, 256)
    p_n = _roll_next(p3).reshape(GV, 256)
    out = proj + w_p * (p_p - proj) + w_n * (p_n - proj) + skip
    if bias is not None:
        out = out + bias
    return _elu(out) if act else out


def _body(x2_ref, tf_ref, freqs_ref, pos_ref, Wt_ref, bt_ref,
          C0_ref, S0_ref, b0_ref,
          C1_ref, S1_ref, b1_ref,
          W2_ref, S2_ref, b2_ref,
          Wn1_ref, bn1_ref, Wn2_ref, bn2_ref, E_ref, E1_ref, out_ref):
    coords = x2_ref[...]
    tf = tf_ref[...]
    ang = tf * freqs_ref[...]
    sincos = jnp.concatenate([jnp.sin(ang), jnp.cos(ang)], axis=1)
    temb = _silu(_mm(sincos, Wt_ref[...]) + bt_ref[...])
    posv = pos_ref[...]

    vidx = jax.lax.broadcasted_iota(jnp.int32, (8, GV), 1) % V
    m0 = (vidx == 0)[:4]
    m63 = (vidx == V - 1)[:4]
    E = E_ref[...]
    E1 = E1_ref[...]

    C0c, C0p, C0t = C0_ref[...][0:2], C0_ref[...][2:6], C0_ref[...][6:134]
    cpart = coords[:, 0:1] * C0c[0:1, :] + coords[:, 1:2] * C0c[1:2, :]
    ppart = _mm(posv, C0p)
    tpart = _mm(temb, C0t) + b0_ref[...]
    big0 = (cpart.reshape(G, V, 512) + ppart[None]
            + tpart[:, None, :]).reshape(GV, 512)

    def logit_t(S):
        nh = S.shape[1]
        lc = _mm_t(S[0:2], coords)
        lp = _mm_t(S[2:6], posv)
        lp = jnp.broadcast_to(lp[:, None, :], (nh, G, V)).reshape(nh, GV)
        lt = _mm_t(S[6:134], temb)
        lt = jnp.broadcast_to(lt[:, :, None], (nh, G, V)).reshape(nh, GV)
        return lc + lp + lt

    st0 = logit_t(S0_ref[...])
    ssT0, stT0 = st0[:4], st0[4:8]
    w_p0, w_n0 = _attn_w(ssT0, stT0, m0, m63)
    h1 = _combine(big0[:, :256], big0[:, 256:512], w_p0, w_n0,
                  None, E, True)

    h1b = h1.astype(BF16)
    big1 = _mm(h1b, C1_ref[...])
    st1 = _mm_t(S1_ref[...], h1b)
    ssT1, stT1 = st1[:4], st1[4:8]
    w_p1, w_n1 = _attn_w(ssT1, stT1, m0, m63)
    h2 = _combine(big1[:, :256], big1[:, 256:512], w_p1, w_n1,
                  b1_ref[...], E, True)

    h2b = h2.astype(BF16)
    big2 = _mm(h2b, W2_ref[...])
    st2 = _mm_t(S2_ref[...], h2b)
    ssT2, stT2 = st2[:1], st2[1:2]
    w_p2, w_n2 = _attn_w(ssT2, stT2, m0[:1], m63[:1])
    h3 = _combine(big2, h2, w_p2, w_n2, b2_ref[...], E1, False)

    hh = _silu(_mm(h3.astype(BF16), Wn1_ref[...]) + bn1_ref[...])
    out_ref[...] = _mm(hh, Wn2_ref[...]) + bn2_ref[...]


def kernel(x, t, W_time, b_time, W0, a_src0, a_dst0, Ws0, bias0,
           W1, a_src1, a_dst1, Ws1, bias1, W2, a_src2, a_dst2, bias2,
           W_nh1, b_nh1, W_nh2, b_nh2):
    N = B * V
    x2 = x.reshape(N, 2)
    tf = t.astype(F32).reshape(B, 1)

    half = TDIM // 2
    freqs = jnp.exp(-jnp.log(10000.0)
                    * jnp.arange(half, dtype=F32) / (half - 1))
    freqs = freqs.reshape(1, half)
    phase = jnp.arange(V, dtype=F32) * (2.0 * np.pi / V)
    pos = jnp.stack([jnp.sin(phase), jnp.cos(phase),
                     jnp.sin(2.0 * phase), jnp.cos(2.0 * phase)], axis=1)

    E = jnp.repeat(jnp.eye(4, dtype=F32), 64, axis=1)
    E1 = jnp.ones((1, 256), dtype=F32)

    def sd(a_s):
        nh = a_s.shape[0]
        if nh == 1:
            return a_s.T
        return (E * a_s.reshape(-1)[None, :]).T

    C0 = jnp.concatenate([W0, Ws0], axis=1)
    S0 = W0 @ jnp.concatenate([sd(a_src0), sd(a_dst0)], axis=1)
    C1 = jnp.concatenate([W1, Ws1], axis=1).astype(BF16)
    S1 = (W1 @ jnp.concatenate([sd(a_src1), sd(a_dst1)],
                               axis=1)).astype(BF16)
    S2 = (W2 @ jnp.concatenate([sd(a_src2), sd(a_dst2)],
                               axis=1)).astype(BF16)

    row = lambda i: (i, 0)
    col = lambda i: (0, i)
    rep = lambda i: (0, 0)
    in_specs = [
        pl.BlockSpec((GV, 2), row),
        pl.BlockSpec((G, 1), row),
        pl.BlockSpec((1, half), rep),
        pl.BlockSpec((V, 4), rep),
        pl.BlockSpec((TDIM, TDIM), rep),
        pl.BlockSpec((1, TDIM), rep),
        pl.BlockSpec((134, 512), rep),
        pl.BlockSpec((134, 8), rep),
        pl.BlockSpec((1, 512), rep),
        pl.BlockSpec((256, 512), rep),
        pl.BlockSpec((256, 8), rep),
        pl.BlockSpec((1, 256), rep),
        pl.BlockSpec((256, 256), rep),
        pl.BlockSpec((256, 2), rep),
        pl.BlockSpec((1, 256), rep),
        pl.BlockSpec((256, 256), rep),
        pl.BlockSpec((1, 256), rep),
        pl.BlockSpec((256, 2), rep),
        pl.BlockSpec((1, 2), rep),
        pl.BlockSpec((4, 256), rep),
        pl.BlockSpec((1, 256), rep),
    ]
    node = pl.pallas_call(
        _body,
        grid=(B // G,),
        in_specs=in_specs,
        out_specs=pl.BlockSpec((GV, 2), row),
        out_shape=jax.ShapeDtypeStruct((N, 2), F32),
    )(x2, tf, freqs, pos, W_time, b_time.reshape(1, TDIM),
      C0, S0, jnp.concatenate([jnp.zeros((1, 256), F32), bias0.reshape(1, 256)], axis=1),
      C1, S1, bias1.reshape(1, 256),
      W2.astype(BF16), S2, bias2.reshape(1, 256),
      W_nh1.astype(BF16), b_nh1.reshape(1, 256),
      W_nh2, b_nh2.reshape(1, 2), E.astype(BF16), E1.astype(BF16))
    return node.reshape(B, 2 * V)


# speedup vs baseline: 1.0317x; 1.0317x over previous
import jax
import jax.numpy as jnp
import numpy as np
from jax.experimental import pallas as pl

B = 1024
V = 64
TDIM = 128
G = 64
GV = G * V
F32 = jnp.float32
BF16 = jnp.bfloat16


def _mm(a, b, out_dtype=F32):
    return jax.lax.dot_general(a, b, (((1,), (0,)), ((), ())),
                               preferred_element_type=out_dtype)


def _mm_t(a, b, out_dtype=F32):
    return jax.lax.dot_general(a, b, (((0,), (1,)), ((), ())),
                               preferred_element_type=out_dtype)


def _mm_tl(a, b, out_dtype=F32):
    return jax.lax.dot_general(a, b, (((0,), (0,)), ((), ())),
                               preferred_element_type=out_dtype)


def _leaky(x):
    return jnp.where(x >= 0, x, 0.2 * x)


def _silu(x):
    return x * jax.lax.logistic(x)


def _elu(x):
    return jnp.where(x > 0, x, jnp.exp(jnp.minimum(x, jnp.zeros_like(x))) - 1)


def _roll_prev(x3):
    return jnp.concatenate([x3[:, V - 1:, :], x3[:, :V - 1, :]], axis=1)


def _roll_next(x3):
    return jnp.concatenate([x3[:, 1:, :], x3[:, :1, :]], axis=1)


def _lroll(x, k):
    return jnp.concatenate([x[:, k:], x[:, :k]], axis=1)


def _attn_w(ssT, stT, m0, m63):
    prv = jnp.where(m0, _lroll(ssT, V - 1), _lroll(ssT, GV - 1))
    nxt = jnp.where(m63, _lroll(ssT, GV - (V - 1)), _lroll(ssT, 1))
    e_s = _leaky(ssT + stT)
    e_p = _leaky(prv + stT)
    e_n = _leaky(nxt + stT)
    m = jnp.maximum(e_s, jnp.maximum(e_p, e_n))
    x_s = jnp.exp(e_s - m)
    x_p = jnp.exp(e_p - m)
    x_n = jnp.exp(e_n - m)
    den = x_s + x_p + x_n + 1e-16
    return x_p / den, x_n / den


def _combine(proj, skip, w_pT, w_nT, bias, E, act):
    w_p = _mm_tl(w_pT.astype(BF16), E)
    w_n = _mm_tl(w_nT.astype(BF16), E)
    p3 = proj.reshape(G, V, 256)
    p_p = _roll_prev(p3).reshape(GV, 256)
    p_n = _roll_next(p3).reshape(GV, 256)
    out = proj + w_p * (p_p - proj) + w_n * (p_n - proj) + skip
    if bias is not None:
        out = out + bias
    return _elu(out) if act else out


def _body(x2_ref, tf_ref, freqs_ref, pos_ref, Wt_ref, bt_ref,
          C0_ref, S0_ref, b0_ref,
          C1_ref, S1_ref, b1_ref,
          W2_ref, S2_ref, b2_ref,
          Wn1_ref, bn1_ref, Wn2_ref, bn2_ref, E_ref, E1_ref, out_ref):
    coords = x2_ref[...]
    tf = tf_ref[...]
    ang = tf * freqs_ref[...]
    sincos = jnp.concatenate([jnp.sin(ang), jnp.cos(ang)], axis=1)
    temb = _silu(_mm(sincos, Wt_ref[...]) + bt_ref[...])
    posv = pos_ref[...]

    vidx = jax.lax.broadcasted_iota(jnp.int32, (8, GV), 1) % V
    m0 = (vidx == 0)[:4]
    m63 = (vidx == V - 1)[:4]
    E = E_ref[...]
    E1 = E1_ref[...]

    C0c, C0p, C0t = C0_ref[...][0:2], C0_ref[...][2:6], C0_ref[...][6:134]
    cpart = coords[:, 0:1] * C0c[0:1, :] + coords[:, 1:2] * C0c[1:2, :]
    ppart = _mm(posv, C0p)
    tpart = _mm(temb, C0t) + b0_ref[...]
    big0 = (cpart.reshape(G, V, 512) + ppart[None]
            + tpart[:, None, :]).reshape(GV, 512)

    def logit_t(S):
        nh = S.shape[1]
        lc = _mm_t(S[0:2], coords)
        lp = _mm_t(S[2:6], posv)
        lp = jnp.broadcast_to(lp[:, None, :], (nh, G, V)).reshape(nh, GV)
        lt = _mm_t(S[6:134], temb)
        lt = jnp.broadcast_to(lt[:, :, None], (nh, G, V)).reshape(nh, GV)
        return lc + lp + lt

    ssT0 = logit_t(S0_ref[...][:, 0:4])
    stT0 = logit_t(S0_ref[...][:, 4:8])
    w_p0, w_n0 = _attn_w(ssT0, stT0, m0, m63)
    h1 = _combine(big0[:, :256], big0[:, 256:512], w_p0, w_n0,
                  None, E, True)

    h1b = h1.astype(BF16)
    big1 = _mm(h1b, C1_ref[...])
    ssT1 = _mm_t(S1_ref[...][:, 0:4], h1b)
    stT1 = _mm_t(S1_ref[...][:, 4:8], h1b)
    w_p1, w_n1 = _attn_w(ssT1, stT1, m0, m63)
    h2 = _combine(big1[:, :256], big1[:, 256:512], w_p1, w_n1,
                  b1_ref[...], E, True)

    h2b = h2.astype(BF16)
    big2 = _mm(h2b, W2_ref[...])
    ssT2 = _mm_t(S2_ref[...][:, 0:1], h2b)
    stT2 = _mm_t(S2_ref[...][:, 1:2], h2b)
    w_p2, w_n2 = _attn_w(ssT2, stT2, m0[:1], m63[:1])
    h3 = _combine(big2, h2, w_p2, w_n2, b2_ref[...], E1, False)

    hh = _silu(_mm(h3.astype(BF16), Wn1_ref[...]) + bn1_ref[...])
    out_ref[...] = _mm(hh, Wn2_ref[...]) + bn2_ref[...]


def kernel(x, t, W_time, b_time, W0, a_src0, a_dst0, Ws0, bias0,
           W1, a_src1, a_dst1, Ws1, bias1, W2, a_src2, a_dst2, bias2,
           W_nh1, b_nh1, W_nh2, b_nh2):
    N = B * V
    x2 = x.reshape(N, 2)
    tf = t.astype(F32).reshape(B, 1)

    half = TDIM // 2
    freqs = jnp.exp(-jnp.log(10000.0)
                    * jnp.arange(half, dtype=F32) / (half - 1))
    freqs = freqs.reshape(1, half)
    phase = jnp.arange(V, dtype=F32) * (2.0 * np.pi / V)
    pos = jnp.stack([jnp.sin(phase), jnp.cos(phase),
                     jnp.sin(2.0 * phase), jnp.cos(2.0 * phase)], axis=1)

    E = jnp.repeat(jnp.eye(4, dtype=F32), 64, axis=1)
    E1 = jnp.ones((1, 256), dtype=F32)

    def sd(a_s):
        nh = a_s.shape[0]
        if nh == 1:
            return a_s.T
        return (E * a_s.reshape(-1)[None, :]).T

    C0 = jnp.concatenate([W0, Ws0], axis=1)
    S0 = W0 @ jnp.concatenate([sd(a_src0), sd(a_dst0)], axis=1)
    C1 = jnp.concatenate([W1, Ws1], axis=1).astype(BF16)
    S1 = (W1 @ jnp.concatenate([sd(a_src1), sd(a_dst1)],
                               axis=1)).astype(BF16)
    S2 = (W2 @ jnp.concatenate([sd(a_src2), sd(a_dst2)],
                               axis=1)).astype(BF16)

    row = lambda i: (i, 0)
    col = lambda i: (0, i)
    rep = lambda i: (0, 0)
    in_specs = [
        pl.BlockSpec((GV, 2), row),
        pl.BlockSpec((G, 1), row),
        pl.BlockSpec((1, half), rep),
        pl.BlockSpec((V, 4), rep),
        pl.BlockSpec((TDIM, TDIM), rep),
        pl.BlockSpec((1, TDIM), rep),
        pl.BlockSpec((134, 512), rep),
        pl.BlockSpec((134, 8), rep),
        pl.BlockSpec((1, 512), rep),
        pl.BlockSpec((256, 512), rep),
        pl.BlockSpec((256, 8), rep),
        pl.BlockSpec((1, 256), rep),
        pl.BlockSpec((256, 256), rep),
        pl.BlockSpec((256, 2), rep),
        pl.BlockSpec((1, 256), rep),
        pl.BlockSpec((256, 256), rep),
        pl.BlockSpec((1, 256), rep),
        pl.BlockSpec((256, 2), rep),
        pl.BlockSpec((1, 2), rep),
        pl.BlockSpec((4, 256), rep),
        pl.BlockSpec((1, 256), rep),
    ]
    node = pl.pallas_call(
        _body,
        grid=(B // G,),
        in_specs=in_specs,
        out_specs=pl.BlockSpec((GV, 2), row),
        out_shape=jax.ShapeDtypeStruct((N, 2), F32),
    )(x2, tf, freqs, pos, W_time, b_time.reshape(1, TDIM),
      C0, S0, jnp.concatenate([jnp.zeros((1, 256), F32), bias0.reshape(1, 256)], axis=1),
      C1, S1, bias1.reshape(1, 256),
      W2.astype(BF16), S2, bias2.reshape(1, 256),
      W_nh1.astype(BF16), b_nh1.reshape(1, 256),
      W_nh2, b_nh2.reshape(1, 2), E.astype(BF16), E1.astype(BF16))
    return node.reshape(B, 2 * V)
